# baseline (speedup 1.0000x reference)
import functools

import jax
import jax.numpy as jnp
from jax import lax
from jax.experimental import pallas as pl
from jax.experimental.pallas import tpu as pltpu
from jax.experimental.pallas import tpu_sc as plsc

R = 5.0
ROWS = 32768
C = 128
NI = 16
NC = 2
NS = 16
NW = NC * NS
T = 256
LOG2E = 1.4426950408889634
L = 16
CV = C // L


def _sc_body(logits_hbm, counts_hbm, out_hbm, counts_v, buf0, buf1, acc,
             sem0, sem1):
    cid = lax.axis_index("c")
    sid = lax.axis_index("s")
    wid = sid * NC + cid

    pltpu.sync_copy(counts_hbm, counts_v)

    cvec = counts_v[pl.ds(0, NI)]
    ends = []
    tot = jnp.int32(0)
    for j in range(NI):
        tot = tot + cvec[j]
        ends.append(tot)
    total = tot
    chunk = (total + NW - 1) // NW
    w_lo = wid * chunk
    w_hi = jnp.minimum(w_lo + chunk, total)

    clamp = lambda x: jnp.minimum(jnp.maximum(x, w_lo), w_hi)
    ab = []
    start = jnp.int32(0)
    for j in range(NI):
        ab.append((clamp(start), clamp(ends[j])))
        start = ends[j]

    zero16 = jnp.zeros((L,), jnp.float32)
    for j in range(NI):
        for cc in range(CV):
            acc[j, pl.ds(cc * L, L)] = zero16

    k_lo = w_lo // T
    k_hi = jnp.where(w_hi > w_lo, (w_hi + T - 1) // T, k_lo)
    nwin = k_hi - k_lo
    bufs = (buf0, buf1)
    sems = (sem0, sem1)

    def dma_start(k, p):
        pltpu.make_async_copy(
            logits_hbm.at[pl.ds(k * T, T), :], bufs[p], sems[p]).start()

    def dma_wait(p):
        pltpu.make_async_copy(
            logits_hbm.at[pl.ds(0, T), :], bufs[p], sems[p]).wait()

    for p in range(2):
        @pl.when(k_lo + p < k_hi)
        def _(p=p):
            dma_start(k_lo + p, p)

    def pair_body(i2, _):
        for p in range(2):
            k = k_lo + i2 * 2 + p

            @pl.when(k < k_hi)
            def _(k=k, p=p):
                dma_wait(p)
                base = k * T
                for j in range(NI):
                    a, b = ab[j]
                    lo = jnp.maximum(a - base, 0)
                    hi = jnp.minimum(b - base, T)

                    @pl.when(hi > lo)
                    def _(j=j, lo=lo, hi=hi, p=p):
                        carry = tuple(
                            acc[j, pl.ds(cc * L, L)] for cc in range(CV))
                        nr = hi - lo

                        def pair_rows(i, cr, lo=lo):
                            r = lo + i * 2
                            outs = []
                            for cc in range(CV):
                                v0 = bufs[p][r, pl.ds(cc * L, L)]
                                v1 = bufs[p][r + 1, pl.ds(cc * L, L)]
                                e = jnp.exp(v0 * R) + jnp.exp(v1 * R)
                                outs.append(cr[cc] + e)
                            return tuple(outs)

                        res = lax.fori_loop(0, nr // 2, pair_rows, carry)

                        def odd_row(i, cr, hi=hi):
                            outs = []
                            for cc in range(CV):
                                v = bufs[p][hi - 1, pl.ds(cc * L, L)]
                                outs.append(cr[cc] + jnp.exp(v * R))
                            return tuple(outs)

                        res = lax.fori_loop(0, nr % 2, odd_row, res)
                        for cc in range(CV):
                            acc[j, pl.ds(cc * L, L)] = res[cc]

                @pl.when(k + 2 < k_hi)
                def _(k=k, p=p):
                    dma_start(k + 2, p)
        return 0

    lax.fori_loop(0, (nwin + 1) // 2, pair_body, 0)

    pltpu.sync_copy(acc, out_hbm.at[wid])


_sc_partial_sums = functools.partial(
    pl.kernel,
    mesh=plsc.VectorSubcoreMesh(core_axis_name="c", subcore_axis_name="s"),
    out_type=jax.ShapeDtypeStruct((NW, NI, C), jnp.float32),
    scratch_types=[
        pltpu.VMEM((NI,), jnp.int32),
        pltpu.VMEM((T, C), jnp.float32),
        pltpu.VMEM((T, C), jnp.float32),
        pltpu.VMEM((NI, C), jnp.float32),
        pltpu.SemaphoreType.DMA,
        pltpu.SemaphoreType.DMA,
    ],
)(_sc_body)


def _tc_finalize_body(partials_ref, counts_ref, out_ref):
    s = jnp.sum(partials_ref[...], axis=0)
    c = counts_ref[...]
    val = (jnp.log(s) - jnp.log(c)) * (1.0 / R)
    out_ref[...] = jnp.where(c > 0, val, 0.0)


def kernel(cell_logits, cell_counts):
    partials = _sc_partial_sums(cell_logits, cell_counts)
    counts_f = cell_counts.astype(jnp.float32).reshape(NI, 1)
    return pl.pallas_call(
        _tc_finalize_body,
        out_shape=jax.ShapeDtypeStruct((NI, C), jnp.float32),
    )(partials, counts_f)


# speedup vs baseline: 1.1270x; 1.1270x over previous
import functools

import jax
import jax.numpy as jnp
from jax import lax
from jax.experimental import pallas as pl
from jax.experimental.pallas import tpu as pltpu
from jax.experimental.pallas import tpu_sc as plsc

R = 5.0
ROWS = 32768
C = 128
NI = 16
NC = 2
NS = 16
NW = NC * NS
T = 256
LOG2E = 1.4426950408889634
L = 16
CV = C // L


def _sc_body(logits_hbm, counts_hbm, out_hbm, counts_v, buf0, buf1, acc,
             idx_v, shared, sem0, sem1):
    cid = lax.axis_index("c")
    sid = lax.axis_index("s")
    wid = sid * NC + cid

    pltpu.sync_copy(counts_hbm, counts_v)

    cvec = counts_v[pl.ds(0, NI)]
    ends = []
    tot = jnp.int32(0)
    for j in range(NI):
        tot = tot + cvec[j]
        ends.append(tot)
    total = tot
    chunk = (total + NW - 1) // NW
    w_lo = wid * chunk
    w_hi = jnp.minimum(w_lo + chunk, total)

    clamp = lambda x: jnp.minimum(jnp.maximum(x, w_lo), w_hi)
    ab = []
    start = jnp.int32(0)
    for j in range(NI):
        ab.append((clamp(start), clamp(ends[j])))
        start = ends[j]

    zero16 = jnp.zeros((L,), jnp.float32)
    for j in range(NI):
        for cc in range(CV):
            acc[j, pl.ds(cc * L, L)] = zero16

    k_lo = w_lo // T
    k_hi = jnp.where(w_hi > w_lo, (w_hi + T - 1) // T, k_lo)
    nwin = k_hi - k_lo
    bufs = (buf0, buf1)
    sems = (sem0, sem1)

    def dma_start(k, p):
        pltpu.make_async_copy(
            logits_hbm.at[pl.ds(k * T, T), :], bufs[p], sems[p]).start()

    def dma_wait(p):
        pltpu.make_async_copy(
            logits_hbm.at[pl.ds(0, T), :], bufs[p], sems[p]).wait()

    for p in range(2):
        @pl.when(k_lo + p < k_hi)
        def _(p=p):
            dma_start(k_lo + p, p)

    def pair_body(i2, _):
        for p in range(2):
            k = k_lo + i2 * 2 + p

            @pl.when(k < k_hi)
            def _(k=k, p=p):
                dma_wait(p)
                base = k * T
                for j in range(NI):
                    a, b = ab[j]
                    lo = jnp.maximum(a - base, 0)
                    hi = jnp.minimum(b - base, T)

                    @pl.when(hi > lo)
                    def _(j=j, lo=lo, hi=hi, p=p):
                        carry = tuple(
                            acc[j, pl.ds(cc * L, L)] for cc in range(CV))

                        def row_body(r, cr):
                            outs = []
                            for cc in range(CV):
                                v = bufs[p][r, pl.ds(cc * L, L)]
                                outs.append(cr[cc] + jnp.exp(v * R))
                            return tuple(outs)

                        res = lax.fori_loop(lo, hi, row_body, carry)
                        for cc in range(CV):
                            acc[j, pl.ds(cc * L, L)] = res[cc]

                @pl.when(k + 2 < k_hi)
                def _(k=k, p=p):
                    dma_start(k + 2, p)
        return 0

    lax.fori_loop(0, (nwin + 1) // 2, pair_body, 0)

    idx_v[pl.ds(0, NI)] = lax.iota(jnp.int32, NI)

    @pl.when(sid == 0)
    def _():
        pltpu.sync_copy(acc, shared)

    plsc.subcore_barrier()

    @pl.when(sid != 0)
    def _():
        pltpu.sync_copy(acc, shared.at[idx_v], add=True)

    plsc.subcore_barrier()

    @pl.when(sid == 0)
    def _():
        pltpu.sync_copy(shared, out_hbm.at[cid])


_sc_partial_sums = functools.partial(
    pl.kernel,
    mesh=plsc.VectorSubcoreMesh(core_axis_name="c", subcore_axis_name="s"),
    out_type=jax.ShapeDtypeStruct((NC, NI, C), jnp.float32),
    scratch_types=[
        pltpu.VMEM((NI,), jnp.int32),
        pltpu.VMEM((T, C), jnp.float32),
        pltpu.VMEM((T, C), jnp.float32),
        pltpu.VMEM((NI, C), jnp.float32),
        pltpu.VMEM((NI,), jnp.int32),
        pltpu.VMEM_SHARED((NI, C), jnp.float32),
        pltpu.SemaphoreType.DMA,
        pltpu.SemaphoreType.DMA,
    ],
)(_sc_body)


def _tc_finalize_body(partials_ref, counts_ref, out_ref):
    s = jnp.sum(partials_ref[...], axis=0)
    c = counts_ref[...]
    val = (jnp.log(s) - jnp.log(c)) * (1.0 / R)
    out_ref[...] = jnp.where(c > 0, val, 0.0)


def kernel(cell_logits, cell_counts):
    partials = _sc_partial_sums(cell_logits, cell_counts)
    counts_f = cell_counts.astype(jnp.float32).reshape(NI, 1)
    return pl.pallas_call(
        _tc_finalize_body,
        out_shape=jax.ShapeDtypeStruct((NI, C), jnp.float32),
    )(partials, counts_f)


# speedup vs baseline: 1.2133x; 1.0766x over previous
import functools

import jax
import jax.numpy as jnp
from jax import lax
from jax.experimental import pallas as pl
from jax.experimental.pallas import tpu as pltpu
from jax.experimental.pallas import tpu_sc as plsc

R = 5.0
ROWS = 32768
C = 128
NI = 16
NC = 2
NS = 16
NW = NC * NS
T = 256
L = 16
CV = C // L
S = 8192
BR = 1024


def _sc_body(logits_hbm, counts_hbm, out_hbm, counts_v, buf0, buf1, acc,
             idx_v, shared, sem0, sem1):
    cid = lax.axis_index("c")
    sid = lax.axis_index("s")
    wid = sid * NC + cid

    pltpu.sync_copy(counts_hbm, counts_v)

    cvec = counts_v[pl.ds(0, NI)]
    ends = []
    tot = jnp.int32(0)
    for j in range(NI):
        tot = tot + cvec[j]
        ends.append(tot)
    total = tot
    total_sc = jnp.maximum(total - S, 0)
    chunk = (total_sc + NW - 1) // NW
    w_lo = S + wid * chunk
    w_hi = jnp.minimum(w_lo + chunk, total)

    clamp = lambda x: jnp.minimum(jnp.maximum(x, w_lo), w_hi)
    ab = []
    start = jnp.int32(0)
    for j in range(NI):
        ab.append((clamp(start), clamp(ends[j])))
        start = ends[j]

    zero16 = jnp.zeros((L,), jnp.float32)
    for j in range(NI):
        for cc in range(CV):
            acc[j, pl.ds(cc * L, L)] = zero16

    k_lo = w_lo // T
    k_hi = jnp.where(w_hi > w_lo, (w_hi + T - 1) // T, k_lo)
    nwin = k_hi - k_lo
    bufs = (buf0, buf1)
    sems = (sem0, sem1)

    def dma_start(k, p):
        pltpu.make_async_copy(
            logits_hbm.at[pl.ds(k * T, T), :], bufs[p], sems[p]).start()

    def dma_wait(p):
        pltpu.make_async_copy(
            logits_hbm.at[pl.ds(0, T), :], bufs[p], sems[p]).wait()

    for p in range(2):
        @pl.when(k_lo + p < k_hi)
        def _(p=p):
            dma_start(k_lo + p, p)

    def pair_body(i2, _):
        for p in range(2):
            k = k_lo + i2 * 2 + p

            @pl.when(k < k_hi)
            def _(k=k, p=p):
                dma_wait(p)
                base = k * T
                for j in range(NI):
                    a, b = ab[j]
                    lo = jnp.maximum(a - base, 0)
                    hi = jnp.minimum(b - base, T)

                    @pl.when(hi > lo)
                    def _(j=j, lo=lo, hi=hi, p=p):
                        carry = tuple(
                            acc[j, pl.ds(cc * L, L)] for cc in range(CV))

                        def row_body(r, cr):
                            outs = []
                            for cc in range(CV):
                                v = bufs[p][r, pl.ds(cc * L, L)]
                                outs.append(cr[cc] + jnp.exp(v * R))
                            return tuple(outs)

                        res = lax.fori_loop(lo, hi, row_body, carry)
                        for cc in range(CV):
                            acc[j, pl.ds(cc * L, L)] = res[cc]

                @pl.when(k + 2 < k_hi)
                def _(k=k, p=p):
                    dma_start(k + 2, p)
        return 0

    lax.fori_loop(0, (nwin + 1) // 2, pair_body, 0)

    idx_v[pl.ds(0, NI)] = lax.iota(jnp.int32, NI)

    @pl.when(sid == 0)
    def _():
        pltpu.sync_copy(acc, shared)

    plsc.subcore_barrier()

    @pl.when(sid != 0)
    def _():
        pltpu.sync_copy(acc, shared.at[idx_v], add=True)

    plsc.subcore_barrier()

    @pl.when(sid == 0)
    def _():
        pltpu.sync_copy(shared, out_hbm.at[cid])


_sc_partial_sums = functools.partial(
    pl.kernel,
    mesh=plsc.VectorSubcoreMesh(core_axis_name="c", subcore_axis_name="s"),
    out_type=jax.ShapeDtypeStruct((NC, NI, C), jnp.float32),
    scratch_types=[
        pltpu.VMEM((NI,), jnp.int32),
        pltpu.VMEM((T, C), jnp.float32),
        pltpu.VMEM((T, C), jnp.float32),
        pltpu.VMEM((NI, C), jnp.float32),
        pltpu.VMEM((NI,), jnp.int32),
        pltpu.VMEM_SHARED((NI, C), jnp.float32),
        pltpu.SemaphoreType.DMA,
        pltpu.SemaphoreType.DMA,
    ],
)(_sc_body)


def _tc_prefix_body(x_ref, counts_ref, out_ref):
    i = pl.program_id(0)

    @pl.when(i == 0)
    def _():
        out_ref[...] = jnp.zeros((NI, C), jnp.float32)

    c = counts_ref[...]
    jj = jax.lax.broadcasted_iota(jnp.int32, (NI, NI), 0)
    kk = jax.lax.broadcasted_iota(jnp.int32, (NI, NI), 1)
    tri = (kk <= jj).astype(jnp.float32)
    ends = jnp.dot(tri, c, preferred_element_type=jnp.float32,
                   precision=jax.lax.Precision.HIGHEST)
    starts = ends - c
    rows = (jax.lax.broadcasted_iota(jnp.int32, (NI, BR), 1)
            .astype(jnp.float32) + (i * BR).astype(jnp.float32))
    m = ((rows >= starts) & (rows < ends)).astype(jnp.float32)
    e = jnp.exp(x_ref[...] * R)
    out_ref[...] += jnp.dot(m, e, preferred_element_type=jnp.float32,
                            precision=jax.lax.Precision.HIGHEST)


def _tc_finalize_body(sc_ref, tc_ref, counts_ref, out_ref):
    s = jnp.sum(sc_ref[...], axis=0) + tc_ref[...]
    c = counts_ref[...]
    val = (jnp.log(s) - jnp.log(c)) * (1.0 / R)
    out_ref[...] = jnp.where(c > 0, val, 0.0)


def kernel(cell_logits, cell_counts):
    counts_f = cell_counts.astype(jnp.float32).reshape(NI, 1)
    sc_partials = _sc_partial_sums(cell_logits, cell_counts)
    tc_partial = pl.pallas_call(
        _tc_prefix_body,
        grid=(S // BR,),
        in_specs=[
            pl.BlockSpec((BR, C), lambda i: (i, 0)),
            pl.BlockSpec((NI, 1), lambda i: (0, 0)),
        ],
        out_specs=pl.BlockSpec((NI, C), lambda i: (0, 0)),
        out_shape=jax.ShapeDtypeStruct((NI, C), jnp.float32),
    )(cell_logits, counts_f)
    return pl.pallas_call(
        _tc_finalize_body,
        out_shape=jax.ShapeDtypeStruct((NI, C), jnp.float32),
    )(sc_partials, tc_partial, counts_f)


# speedup vs baseline: 1.2486x; 1.0291x over previous
import functools

import jax
import jax.numpy as jnp
from jax import lax
from jax.experimental import pallas as pl
from jax.experimental.pallas import tpu as pltpu
from jax.experimental.pallas import tpu_sc as plsc

R = 5.0
ROWS = 32768
C = 128
NI = 16
NC = 2
NS = 16
NW = NC * NS
T = 256
L = 16
CV = C // L
S = 10240
BR = 1024


def _sc_body(logits_hbm, counts_hbm, out_hbm, counts_v, buf0, buf1, acc,
             idx_v, shared, sem0, sem1):
    cid = lax.axis_index("c")
    sid = lax.axis_index("s")
    wid = sid * NC + cid

    pltpu.sync_copy(counts_hbm, counts_v)

    cvec = counts_v[pl.ds(0, NI)]
    ends = []
    tot = jnp.int32(0)
    for j in range(NI):
        tot = tot + cvec[j]
        ends.append(tot)
    total = tot
    total_sc = jnp.maximum(total - S, 0)
    chunk = (total_sc + NW - 1) // NW
    w_lo = S + wid * chunk
    w_hi = jnp.minimum(w_lo + chunk, total)

    clamp = lambda x: jnp.minimum(jnp.maximum(x, w_lo), w_hi)
    ab = []
    start = jnp.int32(0)
    for j in range(NI):
        ab.append((clamp(start), clamp(ends[j])))
        start = ends[j]

    zero16 = jnp.zeros((L,), jnp.float32)
    for j in range(NI):
        for cc in range(CV):
            acc[j, pl.ds(cc * L, L)] = zero16

    k_lo = w_lo // T
    k_hi = jnp.where(w_hi > w_lo, (w_hi + T - 1) // T, k_lo)
    nwin = k_hi - k_lo
    bufs = (buf0, buf1)
    sems = (sem0, sem1)

    def dma_start(k, p):
        pltpu.make_async_copy(
            logits_hbm.at[pl.ds(k * T, T), :], bufs[p], sems[p]).start()

    def dma_wait(p):
        pltpu.make_async_copy(
            logits_hbm.at[pl.ds(0, T), :], bufs[p], sems[p]).wait()

    for p in range(2):
        @pl.when(k_lo + p < k_hi)
        def _(p=p):
            dma_start(k_lo + p, p)

    def pair_body(i2, _):
        for p in range(2):
            k = k_lo + i2 * 2 + p

            @pl.when(k < k_hi)
            def _(k=k, p=p):
                dma_wait(p)
                base = k * T
                for j in range(NI):
                    a, b = ab[j]
                    lo = jnp.maximum(a - base, 0)
                    hi = jnp.minimum(b - base, T)

                    @pl.when(hi > lo)
                    def _(j=j, lo=lo, hi=hi, p=p):
                        carry = tuple(
                            acc[j, pl.ds(cc * L, L)] for cc in range(CV))

                        def row_body(r, cr):
                            outs = []
                            for cc in range(CV):
                                v = bufs[p][r, pl.ds(cc * L, L)]
                                outs.append(cr[cc] + jnp.exp(v * R))
                            return tuple(outs)

                        res = lax.fori_loop(lo, hi, row_body, carry)
                        for cc in range(CV):
                            acc[j, pl.ds(cc * L, L)] = res[cc]

                @pl.when(k + 2 < k_hi)
                def _(k=k, p=p):
                    dma_start(k + 2, p)
        return 0

    lax.fori_loop(0, (nwin + 1) // 2, pair_body, 0)

    idx_v[pl.ds(0, NI)] = lax.iota(jnp.int32, NI)

    @pl.when(sid == 0)
    def _():
        pltpu.sync_copy(acc, shared)

    plsc.subcore_barrier()

    @pl.when(sid != 0)
    def _():
        pltpu.sync_copy(acc, shared.at[idx_v], add=True)

    plsc.subcore_barrier()

    @pl.when(sid == 0)
    def _():
        pltpu.sync_copy(shared, out_hbm.at[cid])


_sc_partial_sums = functools.partial(
    pl.kernel,
    mesh=plsc.VectorSubcoreMesh(core_axis_name="c", subcore_axis_name="s"),
    out_type=jax.ShapeDtypeStruct((NC, NI, C), jnp.float32),
    scratch_types=[
        pltpu.VMEM((NI,), jnp.int32),
        pltpu.VMEM((T, C), jnp.float32),
        pltpu.VMEM((T, C), jnp.float32),
        pltpu.VMEM((NI, C), jnp.float32),
        pltpu.VMEM((NI,), jnp.int32),
        pltpu.VMEM_SHARED((NI, C), jnp.float32),
        pltpu.SemaphoreType.DMA,
        pltpu.SemaphoreType.DMA,
    ],
)(_sc_body)


def _tc_prefix_body(x_ref, counts_ref, out_ref):
    i = pl.program_id(0)

    @pl.when(i == 0)
    def _():
        out_ref[...] = jnp.zeros((NI, C), jnp.float32)

    c = counts_ref[...]
    jj = jax.lax.broadcasted_iota(jnp.int32, (NI, NI), 0)
    kk = jax.lax.broadcasted_iota(jnp.int32, (NI, NI), 1)
    tri = (kk <= jj).astype(jnp.float32)
    ends = jnp.dot(tri, c, preferred_element_type=jnp.float32,
                   precision=jax.lax.Precision.HIGHEST)
    starts = ends - c
    rows = (jax.lax.broadcasted_iota(jnp.int32, (NI, BR), 1)
            .astype(jnp.float32) + (i * BR).astype(jnp.float32))
    m = ((rows >= starts) & (rows < ends)).astype(jnp.float32)
    e = jnp.exp(x_ref[...] * R)
    out_ref[...] += jnp.dot(m, e, preferred_element_type=jnp.float32)


def _tc_finalize_body(sc_ref, tc_ref, counts_ref, out_ref):
    s = jnp.sum(sc_ref[...], axis=0) + tc_ref[...]
    c = counts_ref[...]
    val = (jnp.log(s) - jnp.log(c)) * (1.0 / R)
    out_ref[...] = jnp.where(c > 0, val, 0.0)


def kernel(cell_logits, cell_counts):
    counts_f = cell_counts.astype(jnp.float32).reshape(NI, 1)
    sc_partials = _sc_partial_sums(cell_logits, cell_counts)
    tc_partial = pl.pallas_call(
        _tc_prefix_body,
        grid=(S // BR,),
        in_specs=[
            pl.BlockSpec((BR, C), lambda i: (i, 0)),
            pl.BlockSpec((NI, 1), lambda i: (0, 0)),
        ],
        out_specs=pl.BlockSpec((NI, C), lambda i: (0, 0)),
        out_shape=jax.ShapeDtypeStruct((NI, C), jnp.float32),
    )(cell_logits, counts_f)
    return pl.pallas_call(
        _tc_finalize_body,
        out_shape=jax.ShapeDtypeStruct((NI, C), jnp.float32),
    )(sc_partials, tc_partial, counts_f)


# speedup vs baseline: 1.2737x; 1.0201x over previous
import functools

import jax
import jax.numpy as jnp
from jax import lax
from jax.experimental import pallas as pl
from jax.experimental.pallas import tpu as pltpu
from jax.experimental.pallas import tpu_sc as plsc

R = 5.0
ROWS = 32768
C = 128
NI = 16
NC = 2
NS = 16
NW = NC * NS
T = 256
L = 16
CV = C // L
S = 12288
BR = 1024


def _sc_body(logits_hbm, counts_hbm, out_hbm, counts_v, buf0, buf1, acc,
             idx_v, shared, sem0, sem1):
    cid = lax.axis_index("c")
    sid = lax.axis_index("s")
    wid = sid * NC + cid

    pltpu.sync_copy(counts_hbm, counts_v)

    cvec = counts_v[pl.ds(0, NI)]
    ends = []
    tot = jnp.int32(0)
    for j in range(NI):
        tot = tot + cvec[j]
        ends.append(tot)
    total = tot
    total_sc = jnp.maximum(total - S, 0)
    chunk = (total_sc + NW - 1) // NW
    w_lo = S + wid * chunk
    w_hi = jnp.minimum(w_lo + chunk, total)

    clamp = lambda x: jnp.minimum(jnp.maximum(x, w_lo), w_hi)
    ab = []
    start = jnp.int32(0)
    for j in range(NI):
        ab.append((clamp(start), clamp(ends[j])))
        start = ends[j]

    zero16 = jnp.zeros((L,), jnp.float32)
    for j in range(NI):
        for cc in range(CV):
            acc[j, pl.ds(cc * L, L)] = zero16

    k_lo = w_lo // T
    k_hi = jnp.where(w_hi > w_lo, (w_hi + T - 1) // T, k_lo)
    nwin = k_hi - k_lo
    bufs = (buf0, buf1)
    sems = (sem0, sem1)

    def dma_start(k, p):
        pltpu.make_async_copy(
            logits_hbm.at[pl.ds(k * T, T), :], bufs[p], sems[p]).start()

    def dma_wait(p):
        pltpu.make_async_copy(
            logits_hbm.at[pl.ds(0, T), :], bufs[p], sems[p]).wait()

    for p in range(2):
        @pl.when(k_lo + p < k_hi)
        def _(p=p):
            dma_start(k_lo + p, p)

    def pair_body(i2, _):
        for p in range(2):
            k = k_lo + i2 * 2 + p

            @pl.when(k < k_hi)
            def _(k=k, p=p):
                dma_wait(p)
                base = k * T
                for j in range(NI):
                    a, b = ab[j]
                    lo = jnp.maximum(a - base, 0)
                    hi = jnp.minimum(b - base, T)

                    @pl.when(hi > lo)
                    def _(j=j, lo=lo, hi=hi, p=p):
                        carry = tuple(
                            acc[j, pl.ds(cc * L, L)] for cc in range(CV))

                        def row_body(r, cr):
                            outs = []
                            for cc in range(CV):
                                v = bufs[p][r, pl.ds(cc * L, L)]
                                outs.append(cr[cc] + jnp.exp(v * R))
                            return tuple(outs)

                        res = lax.fori_loop(lo, hi, row_body, carry)
                        for cc in range(CV):
                            acc[j, pl.ds(cc * L, L)] = res[cc]

                @pl.when(k + 2 < k_hi)
                def _(k=k, p=p):
                    dma_start(k + 2, p)
        return 0

    lax.fori_loop(0, (nwin + 1) // 2, pair_body, 0)

    idx_v[pl.ds(0, NI)] = lax.iota(jnp.int32, NI)

    @pl.when(sid == 0)
    def _():
        pltpu.sync_copy(acc, shared)

    plsc.subcore_barrier()

    @pl.when(sid != 0)
    def _():
        pltpu.sync_copy(acc, shared.at[idx_v], add=True)

    plsc.subcore_barrier()

    @pl.when(sid == 0)
    def _():
        pltpu.sync_copy(shared, out_hbm.at[cid])


_sc_partial_sums = functools.partial(
    pl.kernel,
    mesh=plsc.VectorSubcoreMesh(core_axis_name="c", subcore_axis_name="s"),
    out_type=jax.ShapeDtypeStruct((NC, NI, C), jnp.float32),
    scratch_types=[
        pltpu.VMEM((NI,), jnp.int32),
        pltpu.VMEM((T, C), jnp.float32),
        pltpu.VMEM((T, C), jnp.float32),
        pltpu.VMEM((NI, C), jnp.float32),
        pltpu.VMEM((NI,), jnp.int32),
        pltpu.VMEM_SHARED((NI, C), jnp.float32),
        pltpu.SemaphoreType.DMA,
        pltpu.SemaphoreType.DMA,
    ],
)(_sc_body)


def _tc_prefix_body(x_ref, counts_ref, out_ref):
    i = pl.program_id(0)

    @pl.when(i == 0)
    def _():
        out_ref[...] = jnp.zeros((NI, C), jnp.float32)

    c = counts_ref[...]
    jj = jax.lax.broadcasted_iota(jnp.int32, (NI, NI), 0)
    kk = jax.lax.broadcasted_iota(jnp.int32, (NI, NI), 1)
    tri = (kk <= jj).astype(jnp.float32)
    ends = jnp.dot(tri, c, preferred_element_type=jnp.float32,
                   precision=jax.lax.Precision.HIGHEST)
    starts = ends - c
    rows = (jax.lax.broadcasted_iota(jnp.int32, (NI, BR), 1)
            .astype(jnp.float32) + (i * BR).astype(jnp.float32))
    m = ((rows >= starts) & (rows < ends)).astype(jnp.float32)
    e = jnp.exp(x_ref[...] * R)
    out_ref[...] += jnp.dot(m, e, preferred_element_type=jnp.float32)


def _tc_finalize_body(sc_ref, tc_ref, counts_ref, out_ref):
    s = jnp.sum(sc_ref[...], axis=0) + tc_ref[...]
    c = counts_ref[...]
    val = (jnp.log(s) - jnp.log(c)) * (1.0 / R)
    out_ref[...] = jnp.where(c > 0, val, 0.0)


def kernel(cell_logits, cell_counts):
    counts_f = cell_counts.astype(jnp.float32).reshape(NI, 1)
    sc_partials = _sc_partial_sums(cell_logits, cell_counts)
    tc_partial = pl.pallas_call(
        _tc_prefix_body,
        grid=(S // BR,),
        in_specs=[
            pl.BlockSpec((BR, C), lambda i: (i, 0)),
            pl.BlockSpec((NI, 1), lambda i: (0, 0)),
        ],
        out_specs=pl.BlockSpec((NI, C), lambda i: (0, 0)),
        out_shape=jax.ShapeDtypeStruct((NI, C), jnp.float32),
    )(cell_logits, counts_f)
    return pl.pallas_call(
        _tc_finalize_body,
        out_shape=jax.ShapeDtypeStruct((NI, C), jnp.float32),
    )(sc_partials, tc_partial, counts_f)
